# Initial kernel scaffold; baseline (speedup 1.0000x reference)
import jax
import jax.numpy as jnp
from jax.experimental import pallas as pl


def kernel(x, table, anchors_a, anchors_b):
    raise NotImplementedError("write your pallas kernel here")


# speedup vs baseline: 7.8796x; 7.8796x over previous
import functools

import jax
import jax.numpy as jnp
from jax import lax
from jax.experimental import pallas as pl
from jax.experimental.pallas import tpu as pltpu
from jax.experimental.pallas import tpu_sc as plsc

NC = 2
NS = 16
L = 16
NW = NC * NS


@jax.jit
def _lut_sc(x, tab_flat, a_t, b_t):
    B, F = x.shape
    TR, D = tab_flat.shape
    C, T = a_t.shape
    R = TR // T
    b_per_w = B // NW
    CHUNK = 32
    NCH = b_per_w // CHUNK

    mesh = plsc.VectorSubcoreMesh(
        core_axis_name="c", subcore_axis_name="s", num_cores=NC,
        num_subcores=NS)

    @functools.partial(
        pl.kernel,
        mesh=mesh,
        compiler_params=pltpu.CompilerParams(
            use_tc_tiling_on_sc=False, needs_layout_passes=False),
        out_type=jax.ShapeDtypeStruct((B, D), jnp.bfloat16),
        scratch_types=[
            pltpu.VMEM((2, CHUNK, F), jnp.float32),
            pltpu.VMEM((2, T, D), jnp.bfloat16),
            pltpu.VMEM((2, 1, D), jnp.bfloat16),
            pltpu.VMEM((C, T), jnp.int32),
            pltpu.VMEM((C, T), jnp.int32),
            pltpu.SemaphoreType.DMA,
            pltpu.SemaphoreType.DMA,
            pltpu.SemaphoreType.DMA,
        ],
    )
    def k(x_hbm, tab_hbm, a_hbm, b_hbm, out_hbm,
          xc, rows, orow, a_v, b_v, xsem, gsem, osem):
        cid = lax.axis_index("c")
        sid = lax.axis_index("s")
        wid = sid * NC + cid
        base = wid * b_per_w

        pltpu.sync_copy(a_hbm, a_v)
        pltpu.sync_copy(b_hbm, b_v)
        toff = lax.iota(jnp.int32, L) * R

        def fire_gather(tok):
            ch = lax.div(tok, CHUNK)
            csel = jnp.full((L,), lax.rem(ch, 2), dtype=jnp.int32)
            rsel = jnp.full((L,), lax.rem(tok, CHUNK), dtype=jnp.int32)
            idx = jnp.zeros((L,), dtype=jnp.int32)
            for c in range(C):
                av = plsc.load_gather(xc, [csel, rsel, a_v[c, :]])
                bv = plsc.load_gather(xc, [csel, rsel, b_v[c, :]])
                idx = idx | jnp.where(av > bv, jnp.int32(1 << c),
                                      jnp.int32(0))
            pltpu.async_copy(
                tab_hbm.at[idx + toff], rows.at[lax.rem(tok, 2)], gsem)

        pltpu.async_copy(x_hbm.at[pl.ds(base, CHUNK)], xc.at[0], xsem)
        pltpu.make_async_copy(
            x_hbm.at[pl.ds(0, CHUNK)], xc.at[0], xsem).wait()
        pltpu.async_copy(
            x_hbm.at[pl.ds(base + CHUNK, CHUNK)], xc.at[1], xsem)
        fire_gather(0)

        def tok_body(j, _):
            buf = lax.rem(j, 2)

            @pl.when(j + 1 < b_per_w)
            def _():
                nxt = j + 1

                @pl.when(lax.rem(nxt, CHUNK) == 0)
                def _():
                    nch = lax.div(nxt, CHUNK)
                    pltpu.make_async_copy(
                        x_hbm.at[pl.ds(0, CHUNK)], xc.at[lax.rem(nch, 2)],
                        xsem).wait()

                    @pl.when(nch + 1 < NCH)
                    def _():
                        pltpu.async_copy(
                            x_hbm.at[pl.ds(base + (nch + 1) * CHUNK, CHUNK)],
                            xc.at[lax.rem(nch + 1, 2)], xsem)

                fire_gather(nxt)

            pltpu.make_async_copy(
                tab_hbm.at[pl.ds(0, T)], rows.at[buf], gsem).wait()

            @pl.when(j >= 2)
            def _():
                pltpu.make_async_copy(
                    out_hbm.at[pl.ds(0, 1)], orow.at[buf], osem).wait()

            G = 2 * L

            def red_body(f, _):
                sl = pl.ds(f * G, G)
                vs = [rows[buf, r, sl] for r in range(T)]
                while len(vs) > 1:
                    vs = [vs[i] + vs[i + 1] for i in range(0, len(vs), 2)]
                orow[buf, 0, sl] = vs[0]
                return 0

            lax.fori_loop(0, D // G, red_body, 0, unroll=4)

            pltpu.async_copy(
                orow.at[buf], out_hbm.at[pl.ds(base + j, 1)], osem)
            return 0

        lax.fori_loop(0, b_per_w, tok_body, 0)
        pltpu.make_async_copy(out_hbm.at[pl.ds(0, 1)], orow.at[0], osem).wait()
        pltpu.make_async_copy(out_hbm.at[pl.ds(0, 1)], orow.at[1], osem).wait()

    return k(x, tab_flat, a_t, b_t)


def kernel(x, table, anchors_a, anchors_b):
    T, R, D = table.shape
    tab_flat = table.reshape(T * R, D).astype(jnp.bfloat16)
    a_t = anchors_a.T.astype(jnp.int32)
    b_t = anchors_b.T.astype(jnp.int32)
    return _lut_sc(x, tab_flat, a_t, b_t).astype(jnp.float32)


# speedup vs baseline: 9.2063x; 1.1684x over previous
import functools

import jax
import jax.numpy as jnp
from jax import lax
from jax.experimental import pallas as pl
from jax.experimental.pallas import tpu as pltpu
from jax.experimental.pallas import tpu_sc as plsc

NC = 2
NS = 16
L = 16
NW = NC * NS
NBUF = 4
OB = 8


@jax.jit
def _lut_sc(x, tab_flat, a_t, b_t):
    B, F = x.shape
    TR, D = tab_flat.shape
    C, T = a_t.shape
    R = TR // T
    b_per_w = B // NW
    CHUNK = 32
    NCH = b_per_w // CHUNK
    NOB = b_per_w // OB

    mesh = plsc.VectorSubcoreMesh(
        core_axis_name="c", subcore_axis_name="s", num_cores=NC,
        num_subcores=NS)

    @functools.partial(
        pl.kernel,
        mesh=mesh,
        compiler_params=pltpu.CompilerParams(
            use_tc_tiling_on_sc=False, needs_layout_passes=False),
        out_type=jax.ShapeDtypeStruct((B, D), jnp.bfloat16),
        scratch_types=[
            pltpu.VMEM((2, CHUNK, F), jnp.float32),
            pltpu.VMEM((NBUF, T, D), jnp.bfloat16),
            pltpu.VMEM((2, OB, D), jnp.bfloat16),
            pltpu.VMEM((C, T), jnp.int32),
            pltpu.VMEM((C, T), jnp.int32),
            pltpu.SemaphoreType.DMA,
            pltpu.SemaphoreType.DMA,
            pltpu.SemaphoreType.DMA,
        ],
    )
    def k(x_hbm, tab_hbm, a_hbm, b_hbm, out_hbm,
          xc, rows, orow, a_v, b_v, xsem, gsem, osem):
        cid = lax.axis_index("c")
        sid = lax.axis_index("s")
        wid = sid * NC + cid
        base = wid * b_per_w

        pltpu.sync_copy(a_hbm, a_v)
        pltpu.sync_copy(b_hbm, b_v)
        toff = lax.iota(jnp.int32, L) * R

        def fire_gather(tok):
            ch = lax.div(tok, CHUNK)
            csel = jnp.full((L,), lax.rem(ch, 2), dtype=jnp.int32)
            rsel = jnp.full((L,), lax.rem(tok, CHUNK), dtype=jnp.int32)
            idx = jnp.zeros((L,), dtype=jnp.int32)
            for c in range(C):
                av = plsc.load_gather(xc, [csel, rsel, a_v[c, :]])
                bv = plsc.load_gather(xc, [csel, rsel, b_v[c, :]])
                idx = idx | jnp.where(av > bv, jnp.int32(1 << c),
                                      jnp.int32(0))
            pltpu.async_copy(
                tab_hbm.at[idx + toff], rows.at[lax.rem(tok, NBUF)], gsem)

        pltpu.async_copy(x_hbm.at[pl.ds(base, CHUNK)], xc.at[0], xsem)
        pltpu.make_async_copy(
            x_hbm.at[pl.ds(0, CHUNK)], xc.at[0], xsem).wait()
        pltpu.async_copy(
            x_hbm.at[pl.ds(base + CHUNK, CHUNK)], xc.at[1], xsem)
        for t in range(NBUF - 1):
            fire_gather(t)

        def tok_body(j, _):
            buf = lax.rem(j, NBUF)

            @pl.when(j + NBUF - 1 < b_per_w)
            def _():
                nxt = j + NBUF - 1

                @pl.when(lax.rem(nxt, CHUNK) == 0)
                def _():
                    nch = lax.div(nxt, CHUNK)
                    pltpu.make_async_copy(
                        x_hbm.at[pl.ds(0, CHUNK)], xc.at[lax.rem(nch, 2)],
                        xsem).wait()

                    @pl.when(nch + 1 < NCH)
                    def _():
                        pltpu.async_copy(
                            x_hbm.at[pl.ds(base + (nch + 1) * CHUNK, CHUNK)],
                            xc.at[lax.rem(nch + 1, 2)], xsem)

                fire_gather(nxt)

            pltpu.make_async_copy(
                tab_hbm.at[pl.ds(0, T)], rows.at[buf], gsem).wait()

            ob = lax.rem(lax.div(j, OB), 2)
            jo = lax.rem(j, OB)

            @pl.when((jo == 0) & (j >= 2 * OB))
            def _():
                pltpu.make_async_copy(
                    out_hbm.at[pl.ds(0, OB)], orow.at[ob], osem).wait()

            G = 2 * L

            def red_body(f, _):
                sl = pl.ds(f * G, G)
                vs = [rows[buf, r, sl] for r in range(T)]
                while len(vs) > 1:
                    vs = [vs[i] + vs[i + 1] for i in range(0, len(vs), 2)]
                orow[ob, jo, sl] = vs[0]
                return 0

            lax.fori_loop(0, D // G, red_body, 0, unroll=8)

            @pl.when(jo == OB - 1)
            def _():
                pltpu.async_copy(
                    orow.at[ob],
                    out_hbm.at[pl.ds(base + (lax.div(j, OB)) * OB, OB)], osem)
            return 0

        lax.fori_loop(0, b_per_w, tok_body, 0)
        pltpu.make_async_copy(out_hbm.at[pl.ds(0, OB)], orow.at[0], osem).wait()
        pltpu.make_async_copy(out_hbm.at[pl.ds(0, OB)], orow.at[1], osem).wait()

    return k(x, tab_flat, a_t, b_t)


def kernel(x, table, anchors_a, anchors_b):
    T, R, D = table.shape
    tab_flat = table.reshape(T * R, D).astype(jnp.bfloat16)
    a_t = anchors_a.T.astype(jnp.int32)
    b_t = anchors_b.T.astype(jnp.int32)
    return _lut_sc(x, tab_flat, a_t, b_t).astype(jnp.float32)


# speedup vs baseline: 9.3546x; 1.0161x over previous
import functools

import jax
import jax.numpy as jnp
from jax import lax
from jax.experimental import pallas as pl
from jax.experimental.pallas import tpu as pltpu
from jax.experimental.pallas import tpu_sc as plsc

NC = 2
NS = 16
L = 16
NW = NC * NS
NBUF = 8
OB = 8


@jax.jit
def _lut_sc(x, tab_flat, a_t, b_t):
    B, F = x.shape
    TR, D = tab_flat.shape
    C, T = a_t.shape
    R = TR // T
    b_per_w = B // NW
    CHUNK = 16
    NCH = b_per_w // CHUNK
    NOB = b_per_w // OB

    mesh = plsc.VectorSubcoreMesh(
        core_axis_name="c", subcore_axis_name="s", num_cores=NC,
        num_subcores=NS)

    @functools.partial(
        pl.kernel,
        mesh=mesh,
        compiler_params=pltpu.CompilerParams(
            use_tc_tiling_on_sc=False, needs_layout_passes=False),
        out_type=jax.ShapeDtypeStruct((B, D), jnp.bfloat16),
        scratch_types=[
            pltpu.VMEM((2, CHUNK, F), jnp.float32),
            pltpu.VMEM((NBUF, T, D), jnp.bfloat16),
            pltpu.VMEM((2, OB, D), jnp.bfloat16),
            pltpu.VMEM((C, T), jnp.int32),
            pltpu.VMEM((C, T), jnp.int32),
            pltpu.SemaphoreType.DMA,
            pltpu.SemaphoreType.DMA,
            pltpu.SemaphoreType.DMA,
        ],
    )
    def k(x_hbm, tab_hbm, a_hbm, b_hbm, out_hbm,
          xc, rows, orow, a_v, b_v, xsem, gsem, osem):
        cid = lax.axis_index("c")
        sid = lax.axis_index("s")
        wid = sid * NC + cid
        base = wid * b_per_w

        pltpu.sync_copy(a_hbm, a_v)
        pltpu.sync_copy(b_hbm, b_v)
        toff = lax.iota(jnp.int32, L) * R

        def fire_gather(tok):
            ch = lax.div(tok, CHUNK)
            csel = jnp.full((L,), lax.rem(ch, 2), dtype=jnp.int32)
            rsel = jnp.full((L,), lax.rem(tok, CHUNK), dtype=jnp.int32)
            idx = jnp.zeros((L,), dtype=jnp.int32)
            for c in range(C):
                av = plsc.load_gather(xc, [csel, rsel, a_v[c, :]])
                bv = plsc.load_gather(xc, [csel, rsel, b_v[c, :]])
                idx = idx | jnp.where(av > bv, jnp.int32(1 << c),
                                      jnp.int32(0))
            pltpu.async_copy(
                tab_hbm.at[idx + toff], rows.at[lax.rem(tok, NBUF)], gsem)

        pltpu.async_copy(x_hbm.at[pl.ds(base, CHUNK)], xc.at[0], xsem)
        pltpu.make_async_copy(
            x_hbm.at[pl.ds(0, CHUNK)], xc.at[0], xsem).wait()
        pltpu.async_copy(
            x_hbm.at[pl.ds(base + CHUNK, CHUNK)], xc.at[1], xsem)
        for t in range(NBUF - 1):
            fire_gather(t)

        def tok_body(j, _):
            buf = lax.rem(j, NBUF)

            @pl.when(j + NBUF - 1 < b_per_w)
            def _():
                nxt = j + NBUF - 1

                @pl.when(lax.rem(nxt, CHUNK) == 0)
                def _():
                    nch = lax.div(nxt, CHUNK)
                    pltpu.make_async_copy(
                        x_hbm.at[pl.ds(0, CHUNK)], xc.at[lax.rem(nch, 2)],
                        xsem).wait()

                    @pl.when(nch + 1 < NCH)
                    def _():
                        pltpu.async_copy(
                            x_hbm.at[pl.ds(base + (nch + 1) * CHUNK, CHUNK)],
                            xc.at[lax.rem(nch + 1, 2)], xsem)

                fire_gather(nxt)

            pltpu.make_async_copy(
                tab_hbm.at[pl.ds(0, T)], rows.at[buf], gsem).wait()

            ob = lax.rem(lax.div(j, OB), 2)
            jo = lax.rem(j, OB)

            @pl.when((jo == 0) & (j >= 2 * OB))
            def _():
                pltpu.make_async_copy(
                    out_hbm.at[pl.ds(0, OB)], orow.at[ob], osem).wait()

            G = 2 * L

            def red_body(f, _):
                sl = pl.ds(f * G, G)
                vs = [rows[buf, r, sl] for r in range(T)]
                while len(vs) > 1:
                    vs = [vs[i] + vs[i + 1] for i in range(0, len(vs), 2)]
                orow[ob, jo, sl] = vs[0]
                return 0

            lax.fori_loop(0, D // G, red_body, 0, unroll=8)

            @pl.when(jo == OB - 1)
            def _():
                pltpu.async_copy(
                    orow.at[ob],
                    out_hbm.at[pl.ds(base + (lax.div(j, OB)) * OB, OB)], osem)
            return 0

        lax.fori_loop(0, b_per_w, tok_body, 0)
        pltpu.make_async_copy(out_hbm.at[pl.ds(0, OB)], orow.at[0], osem).wait()
        pltpu.make_async_copy(out_hbm.at[pl.ds(0, OB)], orow.at[1], osem).wait()

    return k(x, tab_flat, a_t, b_t)


def kernel(x, table, anchors_a, anchors_b):
    T, R, D = table.shape
    tab_flat = table.reshape(T * R, D).astype(jnp.bfloat16)
    a_t = anchors_a.T.astype(jnp.int32)
    b_t = anchors_b.T.astype(jnp.int32)
    return _lut_sc(x, tab_flat, a_t, b_t).astype(jnp.float32)
